# baseline (speedup 1.0000x reference)
import functools

import jax
import jax.numpy as jnp
import numpy as np
from jax import lax
from jax.experimental import pallas as pl
from jax.experimental.pallas import tpu as pltpu
from jax.experimental.pallas import tpu_sc as plsc

NC = 2
NS = 16
NW = NC * NS
LANES = 16

ROWS = 128
COLS = 32768
ROWS_PER_W = ROWS // NW

_UNROLL = 8
_VECS_PER_ROW = COLS // LANES
_INT_MAX = np.int32(2147483647)


@functools.partial(
    pl.kernel,
    mesh=plsc.VectorSubcoreMesh(core_axis_name="c", subcore_axis_name="s"),
    out_type=jax.ShapeDtypeStruct((NW, LANES), jnp.int32),
    compiler_params=pltpu.CompilerParams(needs_layout_passes=False),
    scratch_types=[
        pltpu.VMEM((COLS,), jnp.float32),
        pltpu.VMEM((COLS,), jnp.float32),
        pltpu.VMEM((LANES,), jnp.int32),
        pltpu.SemaphoreType.DMA,
        pltpu.SemaphoreType.DMA,
    ],
)
def _argmax_sc(x_hbm, out_hbm, buf0, buf1, res_ref, sem0, sem1):
    wid = lax.axis_index("s") * NC + lax.axis_index("c")
    base_row = wid * ROWS_PER_W
    bufs = (buf0, buf1)
    sems = (sem0, sem1)
    lane = lax.iota(jnp.int32, LANES)

    copies = [None] * ROWS_PER_W
    copies[0] = pltpu.async_copy(x_hbm.at[base_row], bufs[0], sems[0])

    res = jnp.zeros((LANES,), jnp.int32)
    for j in range(ROWS_PER_W):
        if j + 1 < ROWS_PER_W:
            copies[j + 1] = pltpu.async_copy(
                x_hbm.at[base_row + j + 1], bufs[(j + 1) % 2], sems[(j + 1) % 2]
            )
        copies[j].wait()
        buf = bufs[j % 2]

        def step(i, carry, buf=buf):
            best, bidx, idx = carry
            for u in range(_UNROLL):
                v = buf[pl.ds((i * _UNROLL + u) * LANES, LANES)]
                m = v > best
                best = jnp.where(m, v, best)
                bidx = jnp.where(m, idx, bidx)
                idx = idx + LANES
            return best, bidx, idx

        init = (
            jnp.full((LANES,), -jnp.inf, jnp.float32),
            jnp.zeros((LANES,), jnp.int32),
            lane,
        )
        best, bidx, _ = lax.fori_loop(0, _VECS_PER_ROW // _UNROLL, step, init)

        row_max = jnp.max(best)
        cand = jnp.where(best == row_max, bidx, _INT_MAX)
        row_arg = jnp.min(cand)
        res = jnp.where(lane == j, row_arg, res)

    res_ref[...] = res
    pltpu.sync_copy(res_ref, out_hbm.at[wid])


def kernel(x):
    out = _argmax_sc(x)
    return out[:, :ROWS_PER_W].reshape(ROWS).astype(jnp.int64)
